# baseline (speedup 1.0000x reference)
import functools

import jax
import jax.numpy as jnp
from jax import lax
from jax.experimental import pallas as pl
from jax.experimental.pallas import tpu as pltpu
from jax.experimental.pallas import tpu_sc as plsc

L = 16
NC = 2
NS = 16
NW = NC * NS

EMBED = 16
HIDDEN = 8
ROWS_PER_FETCH = 128 // EMBED
NWEIGHTS = EMBED * HIDDEN + 2 * HIDDEN + 1


def _scores_kernel(B: int, n_padded: int):
    b_per_w = B // NW
    nblk = b_per_w // L
    nchunk = b_per_w // 128
    mesh = plsc.VectorSubcoreMesh(core_axis_name="c", subcore_axis_name="s")
    cp = pltpu.CompilerParams(
        needs_layout_passes=False, use_tc_tiling_on_sc=False
    )

    @functools.partial(
        pl.kernel,
        mesh=mesh,
        compiler_params=cp,
        out_type=jax.ShapeDtypeStruct((B,), jnp.float32),
        scratch_types=[
            pltpu.VMEM((b_per_w,), jnp.int32),
            *[pltpu.VMEM((128,), jnp.int32) for _ in range(4)],
            pltpu.VMEM((b_per_w, 128), jnp.float32),
            pltpu.VMEM((NWEIGHTS, L), jnp.float32),
            pltpu.VMEM((b_per_w,), jnp.float32),
            pltpu.SemaphoreType.DMA,
        ],
    )
    def k(ids_hbm, table_hbm, w_hbm, out_hbm, idx_v, q0, q1, q2, q3, rows_v,
          w_v, score_v, sem):
        wid = lax.axis_index("s") * NC + lax.axis_index("c")
        base = wid * b_per_w
        pltpu.sync_copy(w_hbm, w_v)
        pltpu.sync_copy(ids_hbm.at[pl.ds(base, b_per_w)], idx_v)

        qs = [q0, q1, q2, q3]
        for j in range(nchunk):
            for i in range(128 // L):
                v = idx_v[pl.ds(j * 128 + i * L, L)]
                qs[j][pl.ds(i * L, L)] = lax.shift_right_logical(v, 3)
        copies = [
            pltpu.async_copy(
                table_hbm.at[qs[j]],
                rows_v.at[pl.ds(j * 128, 128), :],
                sem,
            )
            for j in range(nchunk)
        ]
        for c in copies:
            c.wait()

        lanes = lax.iota(jnp.int32, L)

        @pl.loop(0, nblk)
        def _(i):
            row0 = i * L
            ridx = row0 + lanes
            ids = idx_v[pl.ds(row0, L)]
            col0 = (ids & 7) * EMBED
            cols = [
                plsc.load_gather(rows_v, [ridx, col0 + kk])
                for kk in range(EMBED)
            ]
            score = w_v[EMBED * HIDDEN + 2 * HIDDEN]
            for j in range(HIDDEN):
                acc = w_v[EMBED * HIDDEN + j]
                for kk in range(EMBED):
                    acc = acc + cols[kk] * w_v[kk * HIDDEN + j]
                h = jnp.maximum(acc, 0.0)
                score = score + h * w_v[EMBED * HIDDEN + HIDDEN + j]
            score_v[pl.ds(row0, L)] = score

        pltpu.sync_copy(score_v, out_hbm.at[pl.ds(base, b_per_w)])

    return k


def kernel(title_ids, table, W1, b1, W2, b2):
    B = title_ids.shape[0]
    w1b = jnp.broadcast_to(W1.reshape(EMBED, HIDDEN, 1), (EMBED, HIDDEN, L))
    w1b = w1b.reshape(EMBED * HIDDEN, L)
    b1b = jnp.broadcast_to(b1.reshape(HIDDEN, 1), (HIDDEN, L))
    w2b = jnp.broadcast_to(W2.reshape(HIDDEN, 1), (HIDDEN, L))
    b2b = jnp.broadcast_to(b2.reshape(1, 1), (1, L))
    wall = jnp.concatenate([w1b, b1b, w2b, b2b], axis=0).astype(jnp.float32)

    n, d = table.shape
    table128 = table.reshape(n * d // 128, 128)

    scores = _scores_kernel(B, n * d // 128)(
        title_ids.astype(jnp.int32), table128, wall
    )
    return scores.reshape(B, 1)
